# baseline (speedup 1.0000x reference)
import functools

import jax
import jax.numpy as jnp
from jax import lax
from jax.experimental import pallas as pl
from jax.experimental.pallas import tpu as pltpu
from jax.experimental.pallas import tpu_sc as plsc

N = 10000
E = 160000
D = 256
HALF = 128

NC = 2
NS = 16
CH = 128
NCH_TOT = E // CH
NITER = -(-NCH_TOT // NS)
NB = 3
NOUTER = -(-NITER // NB)
SRT = 624
LRT = N - (NS - 1) * SRT


def _striped_copy(src, dst, s):
    @pl.when(s < NS - 1)
    def _():
        pltpu.sync_copy(src.at[pl.ds(s * SRT, SRT)], dst.at[pl.ds(s * SRT, SRT)])

    @pl.when(s == NS - 1)
    def _():
        pltpu.sync_copy(src.at[pl.ds((NS - 1) * SRT, LRT)],
                        dst.at[pl.ds((NS - 1) * SRT, LRT)])

_MESH = plsc.VectorSubcoreMesh(core_axis_name="c", subcore_axis_name="s")


def _deg_core(ei_row, out_row, s, idx_v, ones_v, acc_sh, isem, ssem,
              zeros_hbm, ones_hbm):
    _striped_copy(zeros_hbm, acc_sh, s)
    pltpu.sync_copy(ones_hbm, ones_v)
    plsc.subcore_barrier()

    def outer(g, carry):
        for b in range(NB):
            i = g * NB + b
            prev_ok = jnp.logical_and(g > 0, (i - NB) * NS + s < NCH_TOT)

            @pl.when(prev_ok)
            def _():
                pltpu.make_async_copy(ones_v, acc_sh.at[idx_v.at[b]],
                                      ssem.at[b]).wait()

            @pl.when(i * NS + s < NCH_TOT)
            def _():
                off = (i * NS + s) * CH
                pltpu.async_copy(ei_row.at[pl.ds(off, CH)], idx_v.at[b],
                                 isem.at[b])

        for b in range(NB):
            i = g * NB + b

            @pl.when(i * NS + s < NCH_TOT)
            def _():
                off = (i * NS + s) * CH
                pltpu.make_async_copy(ei_row.at[pl.ds(off, CH)], idx_v.at[b],
                                      isem.at[b]).wait()
                pltpu.async_copy(ones_v, acc_sh.at[idx_v.at[b]], ssem.at[b],
                                 add=True)

        return carry

    lax.fori_loop(0, NOUTER, outer, 0)
    for b in range(NB):
        i = (NOUTER - 1) * NB + b

        @pl.when(i * NS + s < NCH_TOT)
        def _():
            pltpu.make_async_copy(ones_v, acc_sh.at[idx_v.at[b]],
                                  ssem.at[b]).wait()

    plsc.subcore_barrier()
    _striped_copy(acc_sh, out_row, s)


@functools.partial(
    pl.kernel,
    out_type=(
        jax.ShapeDtypeStruct((N, HALF), jnp.float32),
        jax.ShapeDtypeStruct((N, HALF), jnp.float32),
    ),
    mesh=_MESH,
    scratch_types=[
        pltpu.VMEM((NB, CH), jnp.int32),
        pltpu.VMEM((CH, HALF), jnp.float32),
        pltpu.VMEM_SHARED((N, HALF), jnp.float32),
        pltpu.SemaphoreType.DMA((NB,)),
        pltpu.SemaphoreType.DMA((NB,)),
    ],
)
def _degree_kernel(ei, zeros_hbm, ones_hbm, out_od, out_id, idx_v, ones_v,
                   acc_sh, isem, ssem):
    c = lax.axis_index("c")
    s = lax.axis_index("s")

    @pl.when(c == 0)
    def _():
        _deg_core(ei.at[0], out_od, s, idx_v, ones_v, acc_sh, isem, ssem,
                  zeros_hbm, ones_hbm)

    @pl.when(c == 1)
    def _():
        _deg_core(ei.at[1], out_id, s, idx_v, ones_v, acc_sh, isem, ssem,
                  zeros_hbm, ones_hbm)


def _agg_core(hs_half, out_half, s, ei, src_v, dst_v, rows_v, acc_sh,
              isem, gsem, ssem, zeros_hbm):
    _striped_copy(zeros_hbm, acc_sh, s)
    plsc.subcore_barrier()

    def outer(g, carry):
        for b in range(NB):
            i = g * NB + b
            prev_ok = jnp.logical_and(g > 0, (i - NB) * NS + s < NCH_TOT)

            @pl.when(prev_ok)
            def _():
                pltpu.make_async_copy(rows_v.at[b], acc_sh.at[dst_v.at[b]],
                                      ssem.at[b]).wait()

            @pl.when(i * NS + s < NCH_TOT)
            def _():
                off = (i * NS + s) * CH
                pltpu.async_copy(ei.at[0, pl.ds(off, CH)], src_v.at[b],
                                 isem.at[b])
                pltpu.async_copy(ei.at[1, pl.ds(off, CH)], dst_v.at[b],
                                 isem.at[b])

        for b in range(NB):
            i = g * NB + b

            @pl.when(i * NS + s < NCH_TOT)
            def _():
                off = (i * NS + s) * CH
                pltpu.make_async_copy(ei.at[0, pl.ds(off, CH)], src_v.at[b],
                                      isem.at[b]).wait()
                pltpu.make_async_copy(ei.at[1, pl.ds(off, CH)], dst_v.at[b],
                                      isem.at[b]).wait()
                pltpu.async_copy(hs_half.at[src_v.at[b]], rows_v.at[b],
                                 gsem.at[b])

        for b in range(NB):
            i = g * NB + b

            @pl.when(i * NS + s < NCH_TOT)
            def _():
                pltpu.make_async_copy(hs_half.at[src_v.at[b]], rows_v.at[b],
                                      gsem.at[b]).wait()
                pltpu.async_copy(rows_v.at[b], acc_sh.at[dst_v.at[b]],
                                 ssem.at[b], add=True)

        return carry

    lax.fori_loop(0, NOUTER, outer, 0)
    for b in range(NB):
        i = (NOUTER - 1) * NB + b

        @pl.when(i * NS + s < NCH_TOT)
        def _():
            pltpu.make_async_copy(rows_v.at[b], acc_sh.at[dst_v.at[b]],
                                  ssem.at[b]).wait()

    plsc.subcore_barrier()
    _striped_copy(acc_sh, out_half, s)


@functools.partial(
    pl.kernel,
    out_type=(
        jax.ShapeDtypeStruct((N, HALF), jnp.float32),
        jax.ShapeDtypeStruct((N, HALF), jnp.float32),
    ),
    mesh=_MESH,
    scratch_types=[
        pltpu.VMEM((NB, CH), jnp.int32),
        pltpu.VMEM((NB, CH), jnp.int32),
        pltpu.VMEM((NB, CH, HALF), jnp.float32),
        pltpu.VMEM_SHARED((N, HALF), jnp.float32),
        pltpu.SemaphoreType.DMA((NB,)),
        pltpu.SemaphoreType.DMA((NB,)),
        pltpu.SemaphoreType.DMA((NB,)),
    ],
)
def _agg_kernel(hs_a, hs_b, ei, zeros_hbm, out_a, out_b,
                src_v, dst_v, rows_v, acc_sh, isem, gsem, ssem):
    c = lax.axis_index("c")
    s = lax.axis_index("s")

    @pl.when(c == 0)
    def _():
        _agg_core(hs_a, out_a, s, ei, src_v, dst_v, rows_v, acc_sh,
                  isem, gsem, ssem, zeros_hbm)

    @pl.when(c == 1)
    def _():
        _agg_core(hs_b, out_b, s, ei, src_v, dst_v, rows_v, acc_sh,
                  isem, gsem, ssem, zeros_hbm)


R = 2000


def _norm(deg):
    return jnp.where(deg > 0, lax.rsqrt(jnp.maximum(deg, 1e-12)), 0.0)


def _prep_body(f_ref, od_ref, hsa_ref, hsb_ref):
    onorm = _norm(od_ref[:, :1])
    hs = f_ref[...] * onorm
    hsa_ref[...] = hs[:, :HALF]
    hsb_ref[...] = hs[:, HALF:]


def _prep_call(features, out_deg):
    return pl.pallas_call(
        _prep_body,
        grid=(N // R,),
        in_specs=[
            pl.BlockSpec((R, D), lambda i: (i, 0)),
            pl.BlockSpec((R, HALF), lambda i: (i, 0)),
        ],
        out_specs=[
            pl.BlockSpec((R, HALF), lambda i: (i, 0)),
            pl.BlockSpec((R, HALF), lambda i: (i, 0)),
        ],
        out_shape=[
            jax.ShapeDtypeStruct((N, HALF), jnp.float32),
            jax.ShapeDtypeStruct((N, HALF), jnp.float32),
        ],
    )(features, out_deg)


_DOT = (((1,), (0,)), ((), ()))


def _mid_body(aggA_ref, aggB_ref, h_ref, ind_ref, od_ref, W_ref, b_ref,
              hout_ref, hsa_ref, hsb_ref):
    inorm = _norm(ind_ref[:, :1])
    xa = aggA_ref[...] * inorm
    xb = aggB_ref[...] * inorm
    out = (
        lax.dot_general(xa, W_ref[:HALF, :], _DOT, preferred_element_type=jnp.float32)
        + lax.dot_general(xb, W_ref[HALF:, :], _DOT, preferred_element_type=jnp.float32)
        + b_ref[...] + h_ref[...]
    )
    out = jnp.maximum(out, 0.0)
    hout_ref[...] = out
    hs = out * _norm(od_ref[:, :1])
    hsa_ref[...] = hs[:, :HALF]
    hsb_ref[...] = hs[:, HALF:]


def _mid_call(agg_a, agg_b, h, in_deg, out_deg, W, b):
    return pl.pallas_call(
        _mid_body,
        grid=(N // R,),
        in_specs=[
            pl.BlockSpec((R, HALF), lambda i: (i, 0)),
            pl.BlockSpec((R, HALF), lambda i: (i, 0)),
            pl.BlockSpec((R, D), lambda i: (i, 0)),
            pl.BlockSpec((R, HALF), lambda i: (i, 0)),
            pl.BlockSpec((R, HALF), lambda i: (i, 0)),
            pl.BlockSpec((D, D), lambda i: (0, 0)),
            pl.BlockSpec((1, D), lambda i: (0, 0)),
        ],
        out_specs=[
            pl.BlockSpec((R, D), lambda i: (i, 0)),
            pl.BlockSpec((R, HALF), lambda i: (i, 0)),
            pl.BlockSpec((R, HALF), lambda i: (i, 0)),
        ],
        out_shape=[
            jax.ShapeDtypeStruct((N, D), jnp.float32),
            jax.ShapeDtypeStruct((N, HALF), jnp.float32),
            jax.ShapeDtypeStruct((N, HALF), jnp.float32),
        ],
    )(agg_a, agg_b, h, in_deg, out_deg, W, b)


def _final_body(aggA_ref, aggB_ref, ind_ref, W_ref, b_ref, hout_ref):
    inorm = _norm(ind_ref[:, :1])
    xa = aggA_ref[...] * inorm
    xb = aggB_ref[...] * inorm
    hout_ref[...] = (
        lax.dot_general(xa, W_ref[:HALF, :], _DOT, preferred_element_type=jnp.float32)
        + lax.dot_general(xb, W_ref[HALF:, :], _DOT, preferred_element_type=jnp.float32)
        + b_ref[...]
    )


def _final_call(agg_a, agg_b, in_deg, W, b):
    return pl.pallas_call(
        _final_body,
        grid=(N // R,),
        in_specs=[
            pl.BlockSpec((R, HALF), lambda i: (i, 0)),
            pl.BlockSpec((R, HALF), lambda i: (i, 0)),
            pl.BlockSpec((R, HALF), lambda i: (i, 0)),
            pl.BlockSpec((D, D), lambda i: (0, 0)),
            pl.BlockSpec((1, D), lambda i: (0, 0)),
        ],
        out_specs=pl.BlockSpec((R, D), lambda i: (i, 0)),
        out_shape=jax.ShapeDtypeStruct((N, D), jnp.float32),
    )(agg_a, agg_b, in_deg, W, b)


def kernel(features, edge_index, W1, b1, W2, b2, W3, b3):
    ei = edge_index
    ones_buf = jnp.ones((CH, HALF), jnp.float32)
    zeros_nh = jnp.zeros((N, HALF), jnp.float32)

    out_deg, in_deg = _degree_kernel(ei, zeros_nh, ones_buf)

    b1r = b1.reshape(1, D)
    b2r = b2.reshape(1, D)
    b3r = b3.reshape(1, D)

    hs_a, hs_b = _prep_call(features, out_deg)
    agg_a, agg_b = _agg_kernel(hs_a, hs_b, ei, zeros_nh)
    h1, hs_a, hs_b = _mid_call(agg_a, agg_b, features, in_deg, out_deg, W1, b1r)
    agg_a, agg_b = _agg_kernel(hs_a, hs_b, ei, zeros_nh)
    h2, hs_a, hs_b = _mid_call(agg_a, agg_b, h1, in_deg, out_deg, W2, b2r)
    agg_a, agg_b = _agg_kernel(hs_a, hs_b, ei, zeros_nh)
    h3 = _final_call(agg_a, agg_b, in_deg, W3, b3r)
    return h3


# speedup vs baseline: 1.0002x; 1.0002x over previous
import functools

import jax
import jax.numpy as jnp
from jax import lax
from jax.experimental import pallas as pl
from jax.experimental.pallas import tpu as pltpu
from jax.experimental.pallas import tpu_sc as plsc

N = 10000
E = 160000
D = 256
HALF = 128

NC = 2
NS = 16
CH = 128
NCH_TOT = E // CH
NITER = -(-NCH_TOT // NS)
NB = 3
NOUTER = -(-NITER // NB)
SRT = 624
LRT = N - (NS - 1) * SRT


def _striped_copy(src, dst, s):
    @pl.when(s < NS - 1)
    def _():
        pltpu.sync_copy(src.at[pl.ds(s * SRT, SRT)], dst.at[pl.ds(s * SRT, SRT)])

    @pl.when(s == NS - 1)
    def _():
        pltpu.sync_copy(src.at[pl.ds((NS - 1) * SRT, LRT)],
                        dst.at[pl.ds((NS - 1) * SRT, LRT)])

_MESH = plsc.VectorSubcoreMesh(core_axis_name="c", subcore_axis_name="s")


def _deg_core(ei_row, out_row, s, idx_v, ones_v, acc_sh, isem, ssem,
              zeros_hbm, ones_hbm):
    _striped_copy(zeros_hbm, acc_sh, s)
    pltpu.sync_copy(ones_hbm, ones_v)
    plsc.subcore_barrier()

    def outer(g, carry):
        for b in range(NB):
            i = g * NB + b
            prev_ok = jnp.logical_and(g > 0, (i - NB) * NS + s < NCH_TOT)

            @pl.when(prev_ok)
            def _():
                pltpu.make_async_copy(ones_v, acc_sh.at[idx_v.at[b]],
                                      ssem.at[b]).wait()

            @pl.when(i * NS + s < NCH_TOT)
            def _():
                off = (i * NS + s) * CH
                pltpu.async_copy(ei_row.at[pl.ds(off, CH)], idx_v.at[b],
                                 isem.at[b])

        for b in range(NB):
            i = g * NB + b

            @pl.when(i * NS + s < NCH_TOT)
            def _():
                off = (i * NS + s) * CH
                pltpu.make_async_copy(ei_row.at[pl.ds(off, CH)], idx_v.at[b],
                                      isem.at[b]).wait()
                pltpu.async_copy(ones_v, acc_sh.at[idx_v.at[b]], ssem.at[b],
                                 add=True)

        return carry

    lax.fori_loop(0, NOUTER, outer, 0)
    for b in range(NB):
        i = (NOUTER - 1) * NB + b

        @pl.when(i * NS + s < NCH_TOT)
        def _():
            pltpu.make_async_copy(ones_v, acc_sh.at[idx_v.at[b]],
                                  ssem.at[b]).wait()

    plsc.subcore_barrier()
    _striped_copy(acc_sh, out_row, s)


@functools.partial(
    pl.kernel,
    out_type=(
        jax.ShapeDtypeStruct((N, HALF), jnp.float32),
        jax.ShapeDtypeStruct((N, HALF), jnp.float32),
    ),
    mesh=_MESH,
    scratch_types=[
        pltpu.VMEM((NB, CH), jnp.int32),
        pltpu.VMEM((CH, HALF), jnp.float32),
        pltpu.VMEM_SHARED((N, HALF), jnp.float32),
        pltpu.SemaphoreType.DMA((NB,)),
        pltpu.SemaphoreType.DMA((NB,)),
    ],
)
def _degree_kernel(ei, zeros_hbm, ones_hbm, out_od, out_id, idx_v, ones_v,
                   acc_sh, isem, ssem):
    c = lax.axis_index("c")
    s = lax.axis_index("s")

    @pl.when(c == 0)
    def _():
        _deg_core(ei.at[0], out_od, s, idx_v, ones_v, acc_sh, isem, ssem,
                  zeros_hbm, ones_hbm)

    @pl.when(c == 1)
    def _():
        _deg_core(ei.at[1], out_id, s, idx_v, ones_v, acc_sh, isem, ssem,
                  zeros_hbm, ones_hbm)


def _agg_core(hs_half, out_half, s, eip, idx_v, rows_v, acc_sh,
              isem, gsem, ssem, zeros_hbm):
    _striped_copy(zeros_hbm, acc_sh, s)
    plsc.subcore_barrier()

    def outer(g, carry):
        for b in range(NB):
            i = g * NB + b
            prev_ok = jnp.logical_and(g > 0, (i - NB) * NS + s < NCH_TOT)

            @pl.when(prev_ok)
            def _():
                pltpu.make_async_copy(rows_v.at[b], acc_sh.at[idx_v.at[b, 1]],
                                      ssem.at[b]).wait()

            @pl.when(i * NS + s < NCH_TOT)
            def _():
                pltpu.async_copy(eip.at[i * NS + s], idx_v.at[b], isem.at[b])

        for b in range(NB):
            i = g * NB + b

            @pl.when(i * NS + s < NCH_TOT)
            def _():
                pltpu.make_async_copy(eip.at[i * NS + s], idx_v.at[b],
                                      isem.at[b]).wait()
                pltpu.async_copy(hs_half.at[idx_v.at[b, 0]], rows_v.at[b],
                                 gsem.at[b])

        for b in range(NB):
            i = g * NB + b

            @pl.when(i * NS + s < NCH_TOT)
            def _():
                pltpu.make_async_copy(hs_half.at[idx_v.at[b, 0]], rows_v.at[b],
                                      gsem.at[b]).wait()
                pltpu.async_copy(rows_v.at[b], acc_sh.at[idx_v.at[b, 1]],
                                 ssem.at[b], add=True)

        return carry

    lax.fori_loop(0, NOUTER, outer, 0)
    for b in range(NB):
        i = (NOUTER - 1) * NB + b

        @pl.when(i * NS + s < NCH_TOT)
        def _():
            pltpu.make_async_copy(rows_v.at[b], acc_sh.at[idx_v.at[b, 1]],
                                  ssem.at[b]).wait()

    plsc.subcore_barrier()
    _striped_copy(acc_sh, out_half, s)


@functools.partial(
    pl.kernel,
    out_type=(
        jax.ShapeDtypeStruct((N, HALF), jnp.float32),
        jax.ShapeDtypeStruct((N, HALF), jnp.float32),
    ),
    mesh=_MESH,
    scratch_types=[
        pltpu.VMEM((NB, 2, CH), jnp.int32),
        pltpu.VMEM((NB, CH, HALF), jnp.float32),
        pltpu.VMEM_SHARED((N, HALF), jnp.float32),
        pltpu.SemaphoreType.DMA((NB,)),
        pltpu.SemaphoreType.DMA((NB,)),
        pltpu.SemaphoreType.DMA((NB,)),
    ],
)
def _agg_kernel(hs_a, hs_b, eip, zeros_hbm, out_a, out_b,
                idx_v, rows_v, acc_sh, isem, gsem, ssem):
    c = lax.axis_index("c")
    s = lax.axis_index("s")

    @pl.when(c == 0)
    def _():
        _agg_core(hs_a, out_a, s, eip, idx_v, rows_v, acc_sh,
                  isem, gsem, ssem, zeros_hbm)

    @pl.when(c == 1)
    def _():
        _agg_core(hs_b, out_b, s, eip, idx_v, rows_v, acc_sh,
                  isem, gsem, ssem, zeros_hbm)


R = 2000


def _norm(deg):
    return jnp.where(deg > 0, lax.rsqrt(jnp.maximum(deg, 1e-12)), 0.0)


def _prep_body(f_ref, od_ref, hsa_ref, hsb_ref):
    onorm = _norm(od_ref[:, :1])
    hs = f_ref[...] * onorm
    hsa_ref[...] = hs[:, :HALF]
    hsb_ref[...] = hs[:, HALF:]


def _prep_call(features, out_deg):
    return pl.pallas_call(
        _prep_body,
        grid=(N // R,),
        in_specs=[
            pl.BlockSpec((R, D), lambda i: (i, 0)),
            pl.BlockSpec((R, HALF), lambda i: (i, 0)),
        ],
        out_specs=[
            pl.BlockSpec((R, HALF), lambda i: (i, 0)),
            pl.BlockSpec((R, HALF), lambda i: (i, 0)),
        ],
        out_shape=[
            jax.ShapeDtypeStruct((N, HALF), jnp.float32),
            jax.ShapeDtypeStruct((N, HALF), jnp.float32),
        ],
    )(features, out_deg)


_DOT = (((1,), (0,)), ((), ()))


def _mid_body(aggA_ref, aggB_ref, h_ref, ind_ref, od_ref, W_ref, b_ref,
              hout_ref, hsa_ref, hsb_ref):
    inorm = _norm(ind_ref[:, :1])
    xa = aggA_ref[...] * inorm
    xb = aggB_ref[...] * inorm
    out = (
        lax.dot_general(xa, W_ref[:HALF, :], _DOT, preferred_element_type=jnp.float32)
        + lax.dot_general(xb, W_ref[HALF:, :], _DOT, preferred_element_type=jnp.float32)
        + b_ref[...] + h_ref[...]
    )
    out = jnp.maximum(out, 0.0)
    hout_ref[...] = out
    hs = out * _norm(od_ref[:, :1])
    hsa_ref[...] = hs[:, :HALF]
    hsb_ref[...] = hs[:, HALF:]


def _mid_call(agg_a, agg_b, h, in_deg, out_deg, W, b):
    return pl.pallas_call(
        _mid_body,
        grid=(N // R,),
        in_specs=[
            pl.BlockSpec((R, HALF), lambda i: (i, 0)),
            pl.BlockSpec((R, HALF), lambda i: (i, 0)),
            pl.BlockSpec((R, D), lambda i: (i, 0)),
            pl.BlockSpec((R, HALF), lambda i: (i, 0)),
            pl.BlockSpec((R, HALF), lambda i: (i, 0)),
            pl.BlockSpec((D, D), lambda i: (0, 0)),
            pl.BlockSpec((1, D), lambda i: (0, 0)),
        ],
        out_specs=[
            pl.BlockSpec((R, D), lambda i: (i, 0)),
            pl.BlockSpec((R, HALF), lambda i: (i, 0)),
            pl.BlockSpec((R, HALF), lambda i: (i, 0)),
        ],
        out_shape=[
            jax.ShapeDtypeStruct((N, D), jnp.float32),
            jax.ShapeDtypeStruct((N, HALF), jnp.float32),
            jax.ShapeDtypeStruct((N, HALF), jnp.float32),
        ],
    )(agg_a, agg_b, h, in_deg, out_deg, W, b)


def _final_body(aggA_ref, aggB_ref, ind_ref, W_ref, b_ref, hout_ref):
    inorm = _norm(ind_ref[:, :1])
    xa = aggA_ref[...] * inorm
    xb = aggB_ref[...] * inorm
    hout_ref[...] = (
        lax.dot_general(xa, W_ref[:HALF, :], _DOT, preferred_element_type=jnp.float32)
        + lax.dot_general(xb, W_ref[HALF:, :], _DOT, preferred_element_type=jnp.float32)
        + b_ref[...]
    )


def _final_call(agg_a, agg_b, in_deg, W, b):
    return pl.pallas_call(
        _final_body,
        grid=(N // R,),
        in_specs=[
            pl.BlockSpec((R, HALF), lambda i: (i, 0)),
            pl.BlockSpec((R, HALF), lambda i: (i, 0)),
            pl.BlockSpec((R, HALF), lambda i: (i, 0)),
            pl.BlockSpec((D, D), lambda i: (0, 0)),
            pl.BlockSpec((1, D), lambda i: (0, 0)),
        ],
        out_specs=pl.BlockSpec((R, D), lambda i: (i, 0)),
        out_shape=jax.ShapeDtypeStruct((N, D), jnp.float32),
    )(agg_a, agg_b, in_deg, W, b)


def kernel(features, edge_index, W1, b1, W2, b2, W3, b3):
    ei = edge_index
    ones_buf = jnp.ones((CH, HALF), jnp.float32)
    zeros_nh = jnp.zeros((N, HALF), jnp.float32)

    eip = ei.reshape(2, NCH_TOT, CH).transpose(1, 0, 2)

    out_deg, in_deg = _degree_kernel(ei, zeros_nh, ones_buf)

    b1r = b1.reshape(1, D)
    b2r = b2.reshape(1, D)
    b3r = b3.reshape(1, D)

    hs_a, hs_b = _prep_call(features, out_deg)
    agg_a, agg_b = _agg_kernel(hs_a, hs_b, eip, zeros_nh)
    h1, hs_a, hs_b = _mid_call(agg_a, agg_b, features, in_deg, out_deg, W1, b1r)
    agg_a, agg_b = _agg_kernel(hs_a, hs_b, eip, zeros_nh)
    h2, hs_a, hs_b = _mid_call(agg_a, agg_b, h1, in_deg, out_deg, W2, b2r)
    agg_a, agg_b = _agg_kernel(hs_a, hs_b, eip, zeros_nh)
    h3 = _final_call(agg_a, agg_b, in_deg, W3, b3r)
    return h3


# speedup vs baseline: 1.0017x; 1.0014x over previous
import functools

import jax
import jax.numpy as jnp
from jax import lax
from jax.experimental import pallas as pl
from jax.experimental.pallas import tpu as pltpu
from jax.experimental.pallas import tpu_sc as plsc

N = 10000
E = 160000
D = 256
HALF = 128

NC = 2
NS = 16
CH = 128
NCH_TOT = E // CH
NITER = -(-NCH_TOT // NS)
NB = 3
NOUTER = -(-NITER // NB)
SRT = 624
LRT = N - (NS - 1) * SRT


def _striped_copy(src, dst, s):
    @pl.when(s < NS - 1)
    def _():
        pltpu.sync_copy(src.at[pl.ds(s * SRT, SRT)], dst.at[pl.ds(s * SRT, SRT)])

    @pl.when(s == NS - 1)
    def _():
        pltpu.sync_copy(src.at[pl.ds((NS - 1) * SRT, LRT)],
                        dst.at[pl.ds((NS - 1) * SRT, LRT)])

_MESH = plsc.VectorSubcoreMesh(core_axis_name="c", subcore_axis_name="s")


def _deg_core(ei_row, out_row, s, idx_v, ones_v, acc_sh, isem, ssem,
              zeros_hbm, ones_hbm):
    _striped_copy(zeros_hbm, acc_sh, s)
    pltpu.sync_copy(ones_hbm, ones_v)
    plsc.subcore_barrier()

    def outer(g, carry):
        for b in range(NB):
            i = g * NB + b
            prev_ok = jnp.logical_and(g > 0, (i - NB) * NS + s < NCH_TOT)

            @pl.when(prev_ok)
            def _():
                pltpu.make_async_copy(ones_v, acc_sh.at[idx_v.at[b]],
                                      ssem.at[b]).wait()

            @pl.when(i * NS + s < NCH_TOT)
            def _():
                off = (i * NS + s) * CH
                pltpu.async_copy(ei_row.at[pl.ds(off, CH)], idx_v.at[b],
                                 isem.at[b])

        for b in range(NB):
            i = g * NB + b

            @pl.when(i * NS + s < NCH_TOT)
            def _():
                off = (i * NS + s) * CH
                pltpu.make_async_copy(ei_row.at[pl.ds(off, CH)], idx_v.at[b],
                                      isem.at[b]).wait()
                pltpu.async_copy(ones_v, acc_sh.at[idx_v.at[b]], ssem.at[b],
                                 add=True)

        return carry

    lax.fori_loop(0, NOUTER, outer, 0)
    for b in range(NB):
        i = (NOUTER - 1) * NB + b

        @pl.when(i * NS + s < NCH_TOT)
        def _():
            pltpu.make_async_copy(ones_v, acc_sh.at[idx_v.at[b]],
                                  ssem.at[b]).wait()

    plsc.subcore_barrier()
    _striped_copy(acc_sh, out_row, s)


@functools.partial(
    pl.kernel,
    out_type=(
        jax.ShapeDtypeStruct((N, HALF), jnp.float32),
        jax.ShapeDtypeStruct((N, HALF), jnp.float32),
    ),
    mesh=_MESH,
    scratch_types=[
        pltpu.VMEM((NB, CH), jnp.int32),
        pltpu.VMEM((CH, HALF), jnp.float32),
        pltpu.VMEM_SHARED((N, HALF), jnp.float32),
        pltpu.SemaphoreType.DMA((NB,)),
        pltpu.SemaphoreType.DMA((NB,)),
    ],
)
def _degree_kernel(ei, zeros_hbm, ones_hbm, out_od, out_id, idx_v, ones_v,
                   acc_sh, isem, ssem):
    c = lax.axis_index("c")
    s = lax.axis_index("s")

    @pl.when(c == 0)
    def _():
        _deg_core(ei.at[0], out_od, s, idx_v, ones_v, acc_sh, isem, ssem,
                  zeros_hbm, ones_hbm)

    @pl.when(c == 1)
    def _():
        _deg_core(ei.at[1], out_id, s, idx_v, ones_v, acc_sh, isem, ssem,
                  zeros_hbm, ones_hbm)


def _agg_core(hs_half, out_half, s, eip, idx_v, rows_v, acc_sh,
              isem, gsem, ssem, zeros_hbm):
    _striped_copy(zeros_hbm, acc_sh, s)
    plsc.subcore_barrier()

    def outer(g, carry):
        for b in range(NB):
            i = g * NB + b
            prev_ok = jnp.logical_and(g > 0, (i - NB) * NS + s < NCH_TOT)

            @pl.when(prev_ok)
            def _():
                pltpu.make_async_copy(rows_v.at[b], acc_sh.at[idx_v.at[b, 1]],
                                      ssem.at[b]).wait()

            @pl.when(i * NS + s < NCH_TOT)
            def _():
                pltpu.async_copy(eip.at[i * NS + s], idx_v.at[b], isem.at[b])

        for b in range(NB):
            i = g * NB + b

            @pl.when(i * NS + s < NCH_TOT)
            def _():
                pltpu.make_async_copy(eip.at[i * NS + s], idx_v.at[b],
                                      isem.at[b]).wait()
                pltpu.async_copy(hs_half.at[idx_v.at[b, 0]], rows_v.at[b],
                                 gsem.at[b])

        for b in range(NB):
            i = g * NB + b

            @pl.when(i * NS + s < NCH_TOT)
            def _():
                pltpu.make_async_copy(hs_half.at[idx_v.at[b, 0]], rows_v.at[b],
                                      gsem.at[b]).wait()
                pltpu.async_copy(rows_v.at[b], acc_sh.at[idx_v.at[b, 1]],
                                 ssem.at[b], add=True)

        return carry

    lax.fori_loop(0, NOUTER, outer, 0)
    for b in range(NB):
        i = (NOUTER - 1) * NB + b

        @pl.when(i * NS + s < NCH_TOT)
        def _():
            pltpu.make_async_copy(rows_v.at[b], acc_sh.at[idx_v.at[b, 1]],
                                  ssem.at[b]).wait()

    plsc.subcore_barrier()
    _striped_copy(acc_sh, out_half, s)


@functools.partial(
    pl.kernel,
    out_type=(
        jax.ShapeDtypeStruct((N, HALF), jnp.float32),
        jax.ShapeDtypeStruct((N, HALF), jnp.float32),
    ),
    mesh=_MESH,
    scratch_types=[
        pltpu.VMEM((NB, 2, CH), jnp.int32),
        pltpu.VMEM((NB, CH, HALF), jnp.float32),
        pltpu.VMEM_SHARED((N, HALF), jnp.float32),
        pltpu.SemaphoreType.DMA((NB,)),
        pltpu.SemaphoreType.DMA((NB,)),
        pltpu.SemaphoreType.DMA((NB,)),
    ],
)
def _agg_kernel(hs_a, hs_b, eip, zeros_hbm, out_a, out_b,
                idx_v, rows_v, acc_sh, isem, gsem, ssem):
    c = lax.axis_index("c")
    s = lax.axis_index("s")

    @pl.when(c == 0)
    def _():
        _agg_core(hs_a, out_a, s, eip, idx_v, rows_v, acc_sh,
                  isem, gsem, ssem, zeros_hbm)

    @pl.when(c == 1)
    def _():
        _agg_core(hs_b, out_b, s, eip, idx_v, rows_v, acc_sh,
                  isem, gsem, ssem, zeros_hbm)


R = 2000


def _norm(deg):
    return jnp.where(deg > 0, lax.rsqrt(jnp.maximum(deg, 1e-12)), 0.0)


def _prep_body(f_ref, od_ref, hsa_ref, hsb_ref):
    onorm = _norm(od_ref[:, :1])
    hs = f_ref[...] * onorm
    hsa_ref[...] = hs[:, :HALF]
    hsb_ref[...] = hs[:, HALF:]


def _prep_call(features, out_deg):
    return pl.pallas_call(
        _prep_body,
        grid=(N // R,),
        in_specs=[
            pl.BlockSpec((R, D), lambda i: (i, 0)),
            pl.BlockSpec((R, HALF), lambda i: (i, 0)),
        ],
        out_specs=[
            pl.BlockSpec((R, HALF), lambda i: (i, 0)),
            pl.BlockSpec((R, HALF), lambda i: (i, 0)),
        ],
        out_shape=[
            jax.ShapeDtypeStruct((N, HALF), jnp.float32),
            jax.ShapeDtypeStruct((N, HALF), jnp.float32),
        ],
    )(features, out_deg)


_DOT = (((1,), (0,)), ((), ()))


def _matmul_body(x_ref, W_ref, y_ref):
    y_ref[...] = lax.dot_general(x_ref[...], W_ref[...], _DOT,
                                 preferred_element_type=jnp.float32)


def _matmul_call(x, W):
    return pl.pallas_call(
        _matmul_body,
        grid=(N // R,),
        in_specs=[
            pl.BlockSpec((R, D), lambda i: (i, 0)),
            pl.BlockSpec((D, D), lambda i: (0, 0)),
        ],
        out_specs=pl.BlockSpec((R, D), lambda i: (i, 0)),
        out_shape=jax.ShapeDtypeStruct((N, D), jnp.float32),
    )(x, W)


def _mid_body(aggA_ref, aggB_ref, h_ref, ind_ref, od_ref, W_ref, b_ref,
              hout_ref, hsa_ref, hsb_ref):
    inorm = _norm(ind_ref[:, :1])
    out = jnp.concatenate([aggA_ref[...], aggB_ref[...]], axis=1) * inorm
    out = jnp.maximum(out + b_ref[...] + h_ref[...], 0.0)
    hout_ref[...] = out
    y = lax.dot_general(out, W_ref[...], _DOT,
                        preferred_element_type=jnp.float32)
    hs = y * _norm(od_ref[:, :1])
    hsa_ref[...] = hs[:, :HALF]
    hsb_ref[...] = hs[:, HALF:]


def _mid_call(agg_a, agg_b, h, in_deg, out_deg, Wnext, b):
    return pl.pallas_call(
        _mid_body,
        grid=(N // R,),
        in_specs=[
            pl.BlockSpec((R, HALF), lambda i: (i, 0)),
            pl.BlockSpec((R, HALF), lambda i: (i, 0)),
            pl.BlockSpec((R, D), lambda i: (i, 0)),
            pl.BlockSpec((R, HALF), lambda i: (i, 0)),
            pl.BlockSpec((R, HALF), lambda i: (i, 0)),
            pl.BlockSpec((D, D), lambda i: (0, 0)),
            pl.BlockSpec((1, D), lambda i: (0, 0)),
        ],
        out_specs=[
            pl.BlockSpec((R, D), lambda i: (i, 0)),
            pl.BlockSpec((R, HALF), lambda i: (i, 0)),
            pl.BlockSpec((R, HALF), lambda i: (i, 0)),
        ],
        out_shape=[
            jax.ShapeDtypeStruct((N, D), jnp.float32),
            jax.ShapeDtypeStruct((N, HALF), jnp.float32),
            jax.ShapeDtypeStruct((N, HALF), jnp.float32),
        ],
    )(agg_a, agg_b, h, in_deg, out_deg, Wnext, b)


def _final_body(aggA_ref, aggB_ref, ind_ref, b_ref, hout_ref):
    inorm = _norm(ind_ref[:, :1])
    out = jnp.concatenate([aggA_ref[...], aggB_ref[...]], axis=1) * inorm
    hout_ref[...] = out + b_ref[...]


def _final_call(agg_a, agg_b, in_deg, b):
    return pl.pallas_call(
        _final_body,
        grid=(N // R,),
        in_specs=[
            pl.BlockSpec((R, HALF), lambda i: (i, 0)),
            pl.BlockSpec((R, HALF), lambda i: (i, 0)),
            pl.BlockSpec((R, HALF), lambda i: (i, 0)),
            pl.BlockSpec((1, D), lambda i: (0, 0)),
        ],
        out_specs=pl.BlockSpec((R, D), lambda i: (i, 0)),
        out_shape=jax.ShapeDtypeStruct((N, D), jnp.float32),
    )(agg_a, agg_b, in_deg, b)


def kernel(features, edge_index, W1, b1, W2, b2, W3, b3):
    ei = edge_index
    ones_buf = jnp.ones((CH, HALF), jnp.float32)
    zeros_nh = jnp.zeros((N, HALF), jnp.float32)

    eip = ei.reshape(2, NCH_TOT, CH).transpose(1, 0, 2)

    y1 = _matmul_call(features, W1)
    out_deg, in_deg = _degree_kernel(ei, zeros_nh, ones_buf)

    b1r = b1.reshape(1, D)
    b2r = b2.reshape(1, D)
    b3r = b3.reshape(1, D)

    hs_a, hs_b = _prep_call(y1, out_deg)
    agg_a, agg_b = _agg_kernel(hs_a, hs_b, eip, zeros_nh)
    h1, hs_a, hs_b = _mid_call(agg_a, agg_b, features, in_deg, out_deg, W2, b1r)
    agg_a, agg_b = _agg_kernel(hs_a, hs_b, eip, zeros_nh)
    h2, hs_a, hs_b = _mid_call(agg_a, agg_b, h1, in_deg, out_deg, W3, b2r)
    agg_a, agg_b = _agg_kernel(hs_a, hs_b, eip, zeros_nh)
    h3 = _final_call(agg_a, agg_b, in_deg, b3r)
    return h3
